# baseline (speedup 1.0000x reference)
import functools

import jax
import jax.numpy as jnp
from jax import lax
from jax.experimental import pallas as pl
from jax.experimental.pallas import tpu as pltpu
from jax.experimental.pallas import tpu_sc as plsc

_NSC = 2
_NSUB = 16
_CHUNK = 1024


@jax.jit
def _embed_t(x_t, table_t):
    h, b = x_t.shape
    d, v = table_t.shape
    kpc = d // _NSC
    nj = (h + _NSUB - 1) // _NSUB
    nch = b // _CHUNK
    mesh = plsc.VectorSubcoreMesh(core_axis_name="c", subcore_axis_name="s")

    @functools.partial(
        pl.kernel,
        out_type=jax.ShapeDtypeStruct((h, d, b), jnp.float32),
        mesh=mesh,
        scratch_types=(
            [pltpu.VMEM((_CHUNK,), jnp.int32) for _ in range(2)]
            + [pltpu.VMEM((_CHUNK,), jnp.float32) for _ in range(2)]
            + [pltpu.VMEM_SHARED((v,), jnp.float32) for _ in range(2)]
            + [pltpu.SemaphoreType.DMA((2,)),
               pltpu.SemaphoreType.DMA((2,)),
               pltpu.SemaphoreType.DMA((2,)),
               pltpu.SemaphoreType.DMA((2,))]
        ),
        compiler_params=pltpu.CompilerParams(use_tc_tiling_on_sc=True),
    )
    def kern(xt_hbm, tab_hbm, out_hbm, *refs):
        idx_bufs = refs[:2]
        gbufs = refs[2:4]
        sps = refs[4:6]
        stsem, issem, gsem, osem = refs[6:]
        cid = lax.axis_index("c")
        sid = lax.axis_index("s")
        k0 = cid * kpc

        def idx_fire(hj, c, p):
            pltpu.async_copy(
                xt_hbm.at[hj, pl.ds(c * _CHUNK, _CHUNK)], idx_bufs[p],
                issem.at[p])

        @pl.when(sid == 0)
        def _():
            pltpu.async_copy(tab_hbm.at[k0], sps[0], stsem.at[0])
            pltpu.async_copy(tab_hbm.at[k0 + 1], sps[1], stsem.at[1])

        @pl.loop(0, kpc)
        def _(kk):
            k = k0 + kk

            for stg in range(2):
                @pl.when(jnp.logical_and(sid == 0, kk % 2 == stg))
                def _():
                    pltpu.make_async_copy(
                        tab_hbm.at[k], sps[stg], stsem.at[stg]).wait()
            plsc.subcore_barrier()

            for stg in range(2):
                sp = sps[stg]

                @pl.when(kk % 2 == stg)
                def _():
                    for j in range(nj):
                        hj = sid + j * _NSUB
                        have = hj < h if j * _NSUB + _NSUB > h else True

                        @pl.when(have)
                        def _():
                            idx_fire(hj, 0, 0)
                            idx_fire(hj, 1, 1)

                            @pl.loop(0, nch, step=2)
                            def _(c0):
                                for p in range(2):
                                    c = c0 + p
                                    ods = pl.ds(c * _CHUNK, _CHUNK)
                                    pltpu.make_async_copy(
                                        xt_hbm.at[hj, ods], idx_bufs[p],
                                        issem.at[p]).wait()

                                    if j > 0:
                                        cond = True
                                    else:
                                        cond = jnp.logical_or(
                                            kk > 0, c0 > 0)

                                    @pl.when(cond)
                                    def _():
                                        pltpu.make_async_copy(
                                            gbufs[p],
                                            out_hbm.at[hj, k, ods],
                                            osem.at[p]).wait()
                                    pltpu.async_copy(
                                        sp.at[idx_bufs[p]], gbufs[p],
                                        gsem.at[p])
                                    pltpu.make_async_copy(
                                        sp.at[idx_bufs[p]], gbufs[p],
                                        gsem.at[p]).wait()
                                    pltpu.async_copy(
                                        gbufs[p], out_hbm.at[hj, k, ods],
                                        osem.at[p])

                                    @pl.when(c + 2 < nch)
                                    def _():
                                        idx_fire(hj, c + 2, p)

            plsc.subcore_barrier()

            @pl.when(jnp.logical_and(sid == 0, kk + 2 < kpc))
            def _():
                for stg in range(2):
                    @pl.when(kk % 2 == stg)
                    def _():
                        pltpu.async_copy(
                            tab_hbm.at[k + 2], sps[stg], stsem.at[stg])

        for p in (0, 1):
            pltpu.make_async_copy(
                gbufs[p], out_hbm.at[0, k0, pl.ds(0, _CHUNK)],
                osem.at[p]).wait()

    return kern(x_t, table_t)


def kernel(x, table):
    x_t = x.T.astype(jnp.int32)
    table_t = table.T
    out_t = _embed_t(x_t, table_t)
    return out_t.transpose(2, 0, 1)


# speedup vs baseline: 1.4182x; 1.4182x over previous
import functools

import jax
import jax.numpy as jnp
from jax import lax
from jax.experimental import pallas as pl
from jax.experimental.pallas import tpu as pltpu
from jax.experimental.pallas import tpu_sc as plsc

_NSC = 2
_NSUB = 16


@jax.jit
def _embed_t(x_t, table_t):
    h, b = x_t.shape
    d, v = table_t.shape
    kpc = d // _NSC
    nj = (h + _NSUB - 1) // _NSUB
    mesh = plsc.VectorSubcoreMesh(core_axis_name="c", subcore_axis_name="s")

    @functools.partial(
        pl.kernel,
        out_type=jax.ShapeDtypeStruct((h, d, b), jnp.float32),
        mesh=mesh,
        scratch_types=(
            [pltpu.VMEM((b,), jnp.int32) for _ in range(2)]
            + [pltpu.VMEM((b,), jnp.float32) for _ in range(2)]
            + [pltpu.VMEM_SHARED((v,), jnp.float32)]
            + [pltpu.SemaphoreType.DMA,
               pltpu.SemaphoreType.DMA((2,)),
               pltpu.SemaphoreType.DMA((2,)),
               pltpu.SemaphoreType.DMA((2,))]
        ),
        compiler_params=pltpu.CompilerParams(use_tc_tiling_on_sc=True),
    )
    def kern(xt_hbm, tab_hbm, out_hbm, *refs):
        idx_bufs = refs[:2]
        gbufs = refs[2:4]
        sp = refs[4]
        stsem, issem, gsem, osem = refs[5:]
        cid = lax.axis_index("c")
        sid = lax.axis_index("s")
        k0 = cid * kpc

        @pl.loop(0, kpc)
        def _(kk):
            k = k0 + kk

            for j in range(2):
                pltpu.async_copy(
                    xt_hbm.at[sid + j * _NSUB], idx_bufs[j], issem.at[j])

            @pl.when(sid == 0)
            def _():
                pltpu.async_copy(tab_hbm.at[k], sp, stsem)
                pltpu.make_async_copy(tab_hbm.at[k], sp, stsem).wait()
            plsc.subcore_barrier()

            for j in range(nj):
                p = j % 2
                hj = sid + j * _NSUB
                have = hj < h if j * _NSUB + _NSUB > h else True
                hj2 = hj + 2 * _NSUB

                @pl.when(have)
                def _():
                    pltpu.make_async_copy(
                        xt_hbm.at[hj], idx_bufs[p], issem.at[p]).wait()

                    @pl.when(jnp.logical_or(kk > 0, j >= 2))
                    def _():
                        pltpu.make_async_copy(
                            gbufs[p], out_hbm.at[hj, k], osem.at[p]).wait()
                    pltpu.async_copy(
                        sp.at[idx_bufs[p]], gbufs[p], gsem.at[p])
                    pltpu.make_async_copy(
                        sp.at[idx_bufs[p]], gbufs[p], gsem.at[p]).wait()
                    pltpu.async_copy(gbufs[p], out_hbm.at[hj, k], osem.at[p])
                    if j + 2 < nj:
                        @pl.when(hj2 < h)
                        def _():
                            pltpu.async_copy(
                                xt_hbm.at[hj2], idx_bufs[p], issem.at[p])

            plsc.subcore_barrier()

        for p in (0, 1):
            pltpu.make_async_copy(
                gbufs[p], out_hbm.at[0, k0], osem.at[p]).wait()

    return kern(x_t, table_t)


def kernel(x, table):
    x_t = x.T.astype(jnp.int32)
    table_t = table.T
    out_t = _embed_t(x_t, table_t)
    return out_t.transpose(2, 0, 1)
